# baseline (speedup 1.0000x reference)
import functools

import jax
import jax.numpy as jnp
from jax import lax
from jax.experimental import pallas as pl
from jax.experimental.pallas import tpu as pltpu
from jax.experimental.pallas import tpu_sc as plsc

N = 10000
E = 320000
D = 128
HD = D // 2
NACC = 10240
EB = 128
TB = 160
RP = 16 * TB
EP = RP * EB
TS = NACC // 16
NB = 2
KB = 2
SB = KB * EB
TB2 = TB // KB
RP2 = RP // KB
NQ = 4
HS = TB2 // NQ
PH = HS

_MESH = plsc.VectorSubcoreMesh(core_axis_name="c", subcore_axis_name="s")

_SCRATCH = [
    pltpu.VMEM((HS, SB), jnp.int32),
    pltpu.VMEM((HS, SB), jnp.int32),
]
_SCRATCH += [pltpu.VMEM((SB, HD), jnp.float32) for _ in range(NB)]
_SCRATCH += [
    pltpu.VMEM((SB,), jnp.float32),
    pltpu.VMEM_SHARED((NACC, HD), jnp.float32),
    pltpu.VMEM_SHARED((NACC, HD), jnp.float32),
    pltpu.VMEM_SHARED((NACC,), jnp.float32),
]
_SCRATCH += [pltpu.SemaphoreType.DMA for _ in range(2 * NB + 1)]


@functools.partial(
    pl.kernel,
    out_type=(
        jax.ShapeDtypeStruct((2, 2, NACC, HD), jnp.float32),
        jax.ShapeDtypeStruct((2, 2, NACC), jnp.float32),
    ),
    mesh=_MESH,
    scratch_types=_SCRATCH,
    compiler_params=pltpu.CompilerParams(use_tc_tiling_on_sc=False),
)
def _seg_sum_sc(fcat0, fcat1, src0, dst0, src1, dst1, zrows, zdeg, outp, outd,
                sslab, dslab, *rest):
    rows = rest[:NB]
    ones_v, acc, ftab, dacc = rest[NB], rest[NB + 1], rest[NB + 2], rest[NB + 3]
    gsem = rest[NB + 4:2 * NB + 4]
    ssem = rest[2 * NB + 4:3 * NB + 4]
    dsem = rest[3 * NB + 4]

    c = lax.axis_index("c")
    s = lax.axis_index("s")
    for i in range(SB // 16):
        ones_v[pl.ds(i * 16, 16)] = jnp.ones((16,), jnp.float32)
    base = s * TB2

    def gstart(b, p):
        pltpu.async_copy(ftab.at[sslab.at[p]], rows[b], gsem[b])

    def make_step(deg_here):
        def step(b, p):
            pltpu.make_async_copy(ftab.at[sslab.at[p]], rows[b], gsem[b]).wait()
            pltpu.sync_copy(rows[b], acc.at[dslab.at[p]], add=True)

            @pl.when(deg_here)
            def _():
                pltpu.async_copy(ones_v, dacc.at[dslab.at[p]], dsem, add=True)
        return step

    def ddrain(j, carry):
        pltpu.make_async_copy(ones_v, dacc.at[dslab.at[0]], dsem).wait()
        return carry

    for graph, (fcat, srcb, dstb) in enumerate(
            ((fcat0, src0, dst0), (fcat1, src1, dst1))):
        pltpu.sync_copy(zrows.at[pl.ds(s * TS, TS)], acc.at[pl.ds(s * TS, TS)])
        pltpu.sync_copy(zdeg.at[pl.ds(s * TS, TS)], dacc.at[pl.ds(s * TS, TS)])

        @pl.when(s < 15)
        def _():
            pltpu.sync_copy(fcat.at[pl.ds(c * N + s * TS, TS)],
                            ftab.at[pl.ds(s * TS, TS)])

        @pl.when(s == 15)
        def _():
            pltpu.sync_copy(fcat.at[pl.ds(c * N + 15 * TS, N - 15 * TS)],
                            ftab.at[pl.ds(15 * TS, N - 15 * TS)])

        plsc.subcore_barrier()

        for half in range(TB2 // HS):
            hbase = base + half * HS
            pltpu.sync_copy(srcb.at[pl.ds(hbase, HS)], sslab)
            pltpu.sync_copy(dstb.at[pl.ds(hbase, HS)], dslab)
            deg_here = c == (half // (TB2 // HS // 2))
            step = make_step(deg_here)

            for b in range(NB):
                gstart(b, b)

            def outer(g, carry):
                for b in range(NB):
                    p = g * NB + b
                    step(b, p)
                    gstart(b, p + NB)
                return carry

            lax.fori_loop(0, PH // NB - 1, outer, 0)
            for b in range(NB):
                step(b, PH - NB + b)

            @pl.when(deg_here)
            def _():
                lax.fori_loop(0, PH, ddrain, 0)

        plsc.subcore_barrier()
        pltpu.sync_copy(acc.at[pl.ds(s * TS, TS)],
                        outp.at[graph, c, pl.ds(s * TS, TS)])
        pltpu.sync_copy(dacc.at[pl.ds(s * TS, TS)],
                        outd.at[graph, c, pl.ds(s * TS, TS)])
        if graph == 0:
            plsc.subcore_barrier()


BR = 1000


def _enc_block(f_ref, p_ref, deg_ref, wa_ref, wbt_ref, wbb_ref, o_ref):
    f = f_ref[...]
    deg = jnp.maximum(deg_ref[:, 0:1] + deg_ref[:, 1:2], 1.0)
    acc = jnp.dot(f, wa_ref[...], preferred_element_type=jnp.float32)
    acc += jnp.dot(p_ref[0] / deg, wbt_ref[...], preferred_element_type=jnp.float32)
    acc += jnp.dot(p_ref[1] / deg, wbb_ref[...], preferred_element_type=jnp.float32)
    h = jnp.maximum(acc, 0.0)
    o_ref[0] = h[:, :HD]
    o_ref[1] = h[:, HD:]


def _encoder_tc(feats, partials, deg2, wa_t, wbt, wbb):
    return pl.pallas_call(
        _enc_block,
        grid=(N // BR,),
        in_specs=[
            pl.BlockSpec((BR, D), lambda i: (i, 0)),
            pl.BlockSpec((2, BR, HD), lambda i: (0, i, 0)),
            pl.BlockSpec((BR, 2), lambda i: (i, 0)),
            pl.BlockSpec((D, D), lambda i: (0, 0)),
            pl.BlockSpec((HD, D), lambda i: (0, 0)),
            pl.BlockSpec((HD, D), lambda i: (0, 0)),
        ],
        out_specs=pl.BlockSpec((2, BR, HD), lambda i: (0, i, 0)),
        out_shape=jax.ShapeDtypeStruct((2, N, HD), jnp.float32),
    )(feats, partials, deg2, wa_t, wbt, wbb)


def _final_block(h0_ref, p0_ref, g0_ref, h1_ref, p1_ref, g1_ref,
                 wat_ref, wab_ref, wbt_ref, wbb_ref, am_ref, o_ref):
    def enc(h_ref, p_ref, g_ref):
        deg = jnp.maximum(g_ref[:, 0:1] + g_ref[:, 1:2], 1.0)
        acc = jnp.dot(h_ref[0], wat_ref[...], preferred_element_type=jnp.float32)
        acc += jnp.dot(h_ref[1], wab_ref[...], preferred_element_type=jnp.float32)
        acc += jnp.dot(p_ref[0] / deg, wbt_ref[...],
                       preferred_element_type=jnp.float32)
        acc += jnp.dot(p_ref[1] / deg, wbb_ref[...],
                       preferred_element_type=jnp.float32)
        return jnp.maximum(acc, 0.0)

    e0 = enc(h0_ref, p0_ref, g0_ref)
    e1 = enc(h1_ref, p1_ref, g1_ref)
    am = am_ref[...]
    c0 = jnp.dot(e0, am, preferred_element_type=jnp.float32)
    c1 = jnp.dot(e1, am, preferred_element_type=jnp.float32)

    def lrelu(x):
        return jnp.where(x >= 0, x, 0.2 * x)

    s00 = lrelu(c0[:, 0:1] + c0[:, 1:2])
    s01 = lrelu(c0[:, 0:1] + c1[:, 1:2])
    s10 = lrelu(c1[:, 2:3] + c0[:, 3:4])
    s11 = lrelu(c1[:, 2:3] + c1[:, 3:4])
    m0 = jnp.maximum(s00, s01)
    w00 = jnp.exp(s00 - m0)
    w01 = jnp.exp(s01 - m0)
    o_ref[0] = (w00 * e0 + w01 * e1) / (w00 + w01)
    m1 = jnp.maximum(s10, s11)
    w10 = jnp.exp(s10 - m1)
    w11 = jnp.exp(s11 - m1)
    o_ref[1] = (w10 * e0 + w11 * e1) / (w10 + w11)


def _final_tc(h0, p0, g0, h1, p1, g1, wat, wab, wbt, wbb, am):
    split_spec = pl.BlockSpec((2, BR, HD), lambda i: (0, i, 0))
    deg_spec = pl.BlockSpec((BR, 2), lambda i: (i, 0))
    wa_spec = pl.BlockSpec((HD, D), lambda i: (0, 0))
    return pl.pallas_call(
        _final_block,
        grid=(N // BR,),
        in_specs=[
            split_spec, split_spec, deg_spec,
            split_spec, split_spec, deg_spec,
            wa_spec, wa_spec, wa_spec, wa_spec,
            pl.BlockSpec((D, 4), lambda i: (0, 0)),
        ],
        out_specs=pl.BlockSpec((2, BR, D), lambda i: (0, i, 0)),
        out_shape=jax.ShapeDtypeStruct((2, N, D), jnp.float32),
    )(h0, p0, g0, h1, p1, g1, wat, wab, wbt, wbb, am)


def _prep_edges(ei):
    src = ei[0].astype(jnp.int32)
    dst = ei[1].astype(jnp.int32)
    pad = EP - E
    src = jnp.concatenate([src, jnp.zeros((pad,), jnp.int32)]).reshape(RP2, SB)
    dst = jnp.concatenate([dst, jnp.full((pad,), N, jnp.int32)]).reshape(RP2, SB)
    return src, dst


def _split_cols(x):
    return jnp.concatenate([x[:, :HD], x[:, HD:]], axis=0)


def kernel(nodes, features0, features1, edge_index0, edge_index1, W1, W2, att):
    f0 = features0.astype(jnp.float32)
    f1 = features1.astype(jnp.float32)
    s0, d0 = _prep_edges(edge_index0)
    s1, d1 = _prep_edges(edge_index1)
    zrows = jnp.zeros((NACC, HD), jnp.float32)
    zdeg = jnp.zeros((NACC,), jnp.float32)
    w1a = W1[:, :D].T
    w1bt = W1[:, D:D + HD].T
    w1bb = W1[:, D + HD:].T
    w2at = W2[:, :HD].T
    w2ab = W2[:, HD:D].T
    w2bt = W2[:, D:D + HD].T
    w2bb = W2[:, D + HD:].T
    am = att.astype(jnp.float32).reshape(4, D).T

    p1, g = _seg_sum_sc(_split_cols(f0), _split_cols(f1),
                        s0, d0, s1, d1, zrows, zdeg)
    g0t = g[0].T
    g1t = g[1].T
    h0 = _encoder_tc(f0, p1[0], g0t, w1a, w1bt, w1bb)
    h1 = _encoder_tc(f1, p1[1], g1t, w1a, w1bt, w1bb)
    p2, _ = _seg_sum_sc(h0.reshape(2 * N, HD), h1.reshape(2 * N, HD),
                        s0, d0, s1, d1, zrows, zdeg)
    return _final_tc(h0, p2[0], g0t, h1, p2[1], g1t,
                     w2at, w2ab, w2bt, w2bb, am)


# speedup vs baseline: 1.1562x; 1.1562x over previous
import functools

import jax
import jax.numpy as jnp
from jax import lax
from jax.experimental import pallas as pl
from jax.experimental.pallas import tpu as pltpu
from jax.experimental.pallas import tpu_sc as plsc

N = 10000
E = 320000
D = 128
HD = D // 2
NACC = 10240
EB = 128
TB = 160
RP = 16 * TB
EP = RP * EB
TS = NACC // 16
NB = 2
KB = 2
SB = KB * EB
TB2 = TB // KB
RP2 = RP // KB
NQ = 4
HS = TB2 // NQ
PH = HS

_MESH = plsc.VectorSubcoreMesh(core_axis_name="c", subcore_axis_name="s")

_SCRATCH = [
    pltpu.VMEM((HS, SB), jnp.int32),
    pltpu.VMEM((HS, SB), jnp.int32),
]
_SCRATCH += [pltpu.VMEM((SB, HD), jnp.float32) for _ in range(NB)]
_SCRATCH += [
    pltpu.VMEM((SB,), jnp.float32),
    pltpu.VMEM_SHARED((NACC, HD), jnp.float32),
    pltpu.VMEM_SHARED((NACC, HD), jnp.float32),
    pltpu.VMEM_SHARED((NACC,), jnp.float32),
]
_SCRATCH += [pltpu.SemaphoreType.DMA for _ in range(2 * NB + 1)]


@functools.partial(
    pl.kernel,
    out_type=(
        jax.ShapeDtypeStruct((2, NACC, HD), jnp.float32),
        jax.ShapeDtypeStruct((2, NACC), jnp.float32),
    ),
    mesh=_MESH,
    scratch_types=_SCRATCH,
    compiler_params=pltpu.CompilerParams(use_tc_tiling_on_sc=False),
)
def _seg_sum_sc(fcat, srcb, dstb, zrows, zdeg, outp, outd,
                sslab, dslab, *rest):
    rows = rest[:NB]
    ones_v, acc, ftab, dacc = rest[NB], rest[NB + 1], rest[NB + 2], rest[NB + 3]
    gsem = rest[NB + 4:2 * NB + 4]
    ssem = rest[2 * NB + 4:3 * NB + 4]
    dsem = rest[3 * NB + 4]

    c = lax.axis_index("c")
    s = lax.axis_index("s")
    for i in range(SB // 16):
        ones_v[pl.ds(i * 16, 16)] = jnp.ones((16,), jnp.float32)
    base = s * TB2

    def gstart(b, p):
        pltpu.async_copy(ftab.at[sslab.at[p]], rows[b], gsem[b])

    def make_step(deg_here):
        def step(b, p):
            pltpu.make_async_copy(ftab.at[sslab.at[p]], rows[b], gsem[b]).wait()
            pltpu.sync_copy(rows[b], acc.at[dslab.at[p]], add=True)

            @pl.when(deg_here)
            def _():
                pltpu.async_copy(ones_v, dacc.at[dslab.at[p]], dsem, add=True)
        return step

    def ddrain(j, carry):
        pltpu.make_async_copy(ones_v, dacc.at[dslab.at[0]], dsem).wait()
        return carry

    pltpu.sync_copy(zrows.at[pl.ds(s * TS, TS)], acc.at[pl.ds(s * TS, TS)])
    pltpu.sync_copy(zdeg.at[pl.ds(s * TS, TS)], dacc.at[pl.ds(s * TS, TS)])

    @pl.when(s < 15)
    def _():
        pltpu.sync_copy(fcat.at[pl.ds(c * N + s * TS, TS)],
                        ftab.at[pl.ds(s * TS, TS)])

    @pl.when(s == 15)
    def _():
        pltpu.sync_copy(fcat.at[pl.ds(c * N + 15 * TS, N - 15 * TS)],
                        ftab.at[pl.ds(15 * TS, N - 15 * TS)])

    plsc.subcore_barrier()

    for half in range(TB2 // HS):
        hbase = base + half * HS
        pltpu.sync_copy(srcb.at[pl.ds(hbase, HS)], sslab)
        pltpu.sync_copy(dstb.at[pl.ds(hbase, HS)], dslab)
        deg_here = c == (half // (TB2 // HS // 2))
        step = make_step(deg_here)

        for b in range(NB):
            gstart(b, b)

        def outer(g, carry):
            for b in range(NB):
                p = g * NB + b
                step(b, p)
                gstart(b, p + NB)
            return carry

        lax.fori_loop(0, PH // NB - 1, outer, 0)
        for b in range(NB):
            step(b, PH - NB + b)

        @pl.when(deg_here)
        def _():
            lax.fori_loop(0, PH, ddrain, 0)

    plsc.subcore_barrier()
    pltpu.sync_copy(acc.at[pl.ds(s * TS, TS)], outp.at[c, pl.ds(s * TS, TS)])
    pltpu.sync_copy(dacc.at[pl.ds(s * TS, TS)], outd.at[c, pl.ds(s * TS, TS)])


BR = 1000


def _enc_block(f_ref, p_ref, deg_ref, wa_ref, wbt_ref, wbb_ref, o_ref):
    f = f_ref[...]
    deg = jnp.maximum(deg_ref[:, 0:1] + deg_ref[:, 1:2], 1.0)
    acc = jnp.dot(f, wa_ref[...], preferred_element_type=jnp.float32)
    acc += jnp.dot(p_ref[0] / deg, wbt_ref[...], preferred_element_type=jnp.float32)
    acc += jnp.dot(p_ref[1] / deg, wbb_ref[...], preferred_element_type=jnp.float32)
    h = jnp.maximum(acc, 0.0)
    o_ref[0] = h[:, :HD]
    o_ref[1] = h[:, HD:]


def _encoder_tc(feats, partials, deg2, wa_t, wbt, wbb):
    return pl.pallas_call(
        _enc_block,
        grid=(N // BR,),
        in_specs=[
            pl.BlockSpec((BR, D), lambda i: (i, 0)),
            pl.BlockSpec((2, BR, HD), lambda i: (0, i, 0)),
            pl.BlockSpec((BR, 2), lambda i: (i, 0)),
            pl.BlockSpec((D, D), lambda i: (0, 0)),
            pl.BlockSpec((HD, D), lambda i: (0, 0)),
            pl.BlockSpec((HD, D), lambda i: (0, 0)),
        ],
        out_specs=pl.BlockSpec((2, BR, HD), lambda i: (0, i, 0)),
        out_shape=jax.ShapeDtypeStruct((2, N, HD), jnp.float32),
    )(feats, partials, deg2, wa_t, wbt, wbb)


def _final_block(h0_ref, p0_ref, g0_ref, h1_ref, p1_ref, g1_ref,
                 wat_ref, wab_ref, wbt_ref, wbb_ref, am_ref, o_ref):
    def enc(h_ref, p_ref, g_ref):
        deg = jnp.maximum(g_ref[:, 0:1] + g_ref[:, 1:2], 1.0)
        acc = jnp.dot(h_ref[0], wat_ref[...], preferred_element_type=jnp.float32)
        acc += jnp.dot(h_ref[1], wab_ref[...], preferred_element_type=jnp.float32)
        acc += jnp.dot(p_ref[0] / deg, wbt_ref[...],
                       preferred_element_type=jnp.float32)
        acc += jnp.dot(p_ref[1] / deg, wbb_ref[...],
                       preferred_element_type=jnp.float32)
        return jnp.maximum(acc, 0.0)

    e0 = enc(h0_ref, p0_ref, g0_ref)
    e1 = enc(h1_ref, p1_ref, g1_ref)
    am = am_ref[...]
    c0 = jnp.dot(e0, am, preferred_element_type=jnp.float32)
    c1 = jnp.dot(e1, am, preferred_element_type=jnp.float32)

    def lrelu(x):
        return jnp.where(x >= 0, x, 0.2 * x)

    s00 = lrelu(c0[:, 0:1] + c0[:, 1:2])
    s01 = lrelu(c0[:, 0:1] + c1[:, 1:2])
    s10 = lrelu(c1[:, 2:3] + c0[:, 3:4])
    s11 = lrelu(c1[:, 2:3] + c1[:, 3:4])
    m0 = jnp.maximum(s00, s01)
    w00 = jnp.exp(s00 - m0)
    w01 = jnp.exp(s01 - m0)
    o_ref[0] = (w00 * e0 + w01 * e1) / (w00 + w01)
    m1 = jnp.maximum(s10, s11)
    w10 = jnp.exp(s10 - m1)
    w11 = jnp.exp(s11 - m1)
    o_ref[1] = (w10 * e0 + w11 * e1) / (w10 + w11)


def _final_tc(h0, p0, g0, h1, p1, g1, wat, wab, wbt, wbb, am):
    split_spec = pl.BlockSpec((2, BR, HD), lambda i: (0, i, 0))
    deg_spec = pl.BlockSpec((BR, 2), lambda i: (i, 0))
    wa_spec = pl.BlockSpec((HD, D), lambda i: (0, 0))
    return pl.pallas_call(
        _final_block,
        grid=(N // BR,),
        in_specs=[
            split_spec, split_spec, deg_spec,
            split_spec, split_spec, deg_spec,
            wa_spec, wa_spec, wa_spec, wa_spec,
            pl.BlockSpec((D, 4), lambda i: (0, 0)),
        ],
        out_specs=pl.BlockSpec((2, BR, D), lambda i: (0, i, 0)),
        out_shape=jax.ShapeDtypeStruct((2, N, D), jnp.float32),
    )(h0, p0, g0, h1, p1, g1, wat, wab, wbt, wbb, am)


def _prep_edges(ei):
    src = ei[0].astype(jnp.int32)
    dst = ei[1].astype(jnp.int32)
    pad = EP - E
    src = jnp.concatenate([src, jnp.zeros((pad,), jnp.int32)]).reshape(RP2, SB)
    dst = jnp.concatenate([dst, jnp.full((pad,), N, jnp.int32)]).reshape(RP2, SB)
    return src, dst


def _split_cols(x):
    return jnp.concatenate([x[:, :HD], x[:, HD:]], axis=0)


def kernel(nodes, features0, features1, edge_index0, edge_index1, W1, W2, att):
    f0 = features0.astype(jnp.float32)
    f1 = features1.astype(jnp.float32)
    s0, d0 = _prep_edges(edge_index0)
    s1, d1 = _prep_edges(edge_index1)
    zrows = jnp.zeros((NACC, HD), jnp.float32)
    zdeg = jnp.zeros((NACC,), jnp.float32)
    w1a = W1[:, :D].T
    w1bt = W1[:, D:D + HD].T
    w1bb = W1[:, D + HD:].T
    w2at = W2[:, :HD].T
    w2ab = W2[:, HD:D].T
    w2bt = W2[:, D:D + HD].T
    w2bb = W2[:, D + HD:].T
    am = att.astype(jnp.float32).reshape(4, D).T

    def graph(feats, src, dst):
        p1, g = _seg_sum_sc(_split_cols(feats), src, dst, zrows, zdeg)
        g2 = g.T
        hs = _encoder_tc(feats, p1, g2, w1a, w1bt, w1bb)
        p2, _ = _seg_sum_sc(hs.reshape(2 * N, HD), src, dst, zrows, zdeg)
        return hs, p2, g2

    h0, p20, g0t = graph(f0, s0, d0)
    h1, p21, g1t = graph(f1, s1, d1)
    return _final_tc(h0, p20, g0t, h1, p21, g1t,
                     w2at, w2ab, w2bt, w2bb, am)


# speedup vs baseline: 1.1595x; 1.0028x over previous
import functools

import jax
import jax.numpy as jnp
from jax import lax
from jax.experimental import pallas as pl
from jax.experimental.pallas import tpu as pltpu
from jax.experimental.pallas import tpu_sc as plsc

N = 10000
E = 320000
D = 128
HD = D // 2
NACC = 10240
EB = 128
TB = 160
RP = 16 * TB
EP = RP * EB
TS = NACC // 16
NB = 2
KB = 2
SB = KB * EB
TB2 = TB // KB
RP2 = RP // KB
NQ = 4
HS = TB2 // NQ
PH = HS

_MESH = plsc.VectorSubcoreMesh(core_axis_name="c", subcore_axis_name="s")

_SCRATCH = [
    pltpu.VMEM((HS, SB), jnp.int32),
    pltpu.VMEM((HS, SB), jnp.int32),
]
_SCRATCH += [pltpu.VMEM((SB, HD), jnp.float32) for _ in range(NB)]
_SCRATCH += [
    pltpu.VMEM((SB,), jnp.float32),
    pltpu.VMEM_SHARED((NACC, HD), jnp.float32),
    pltpu.VMEM_SHARED((NACC, HD), jnp.float32),
    pltpu.VMEM_SHARED((NACC,), jnp.float32),
]
_SCRATCH += [pltpu.SemaphoreType.DMA for _ in range(NB + 1)]


@functools.partial(
    pl.kernel,
    out_type=(
        jax.ShapeDtypeStruct((2, NACC, HD), jnp.float32),
        jax.ShapeDtypeStruct((2, NACC), jnp.float32),
    ),
    mesh=_MESH,
    scratch_types=_SCRATCH,
    compiler_params=pltpu.CompilerParams(use_tc_tiling_on_sc=False),
)
def _seg_sum_sc(fcat, srcb, dstb, zrows, zdeg, outp, outd,
                sslab, dslab, *rest):
    rows = rest[:NB]
    ones_v, acc, ftab, dacc = rest[NB], rest[NB + 1], rest[NB + 2], rest[NB + 3]
    gsem = rest[NB + 4:2 * NB + 4]
    dsem = rest[2 * NB + 4]

    c = lax.axis_index("c")
    s = lax.axis_index("s")
    for i in range(SB // 16):
        ones_v[pl.ds(i * 16, 16)] = jnp.ones((16,), jnp.float32)
    base = s * TB2

    def gstart(b, p):
        pltpu.async_copy(ftab.at[sslab.at[p]], rows[b], gsem[b])

    def make_step(deg_here):
        def step(b, p):
            pltpu.make_async_copy(ftab.at[sslab.at[p]], rows[b], gsem[b]).wait()
            pltpu.sync_copy(rows[b], acc.at[dslab.at[p]], add=True)

            @pl.when(deg_here)
            def _():
                pltpu.async_copy(ones_v, dacc.at[dslab.at[p]], dsem, add=True)
        return step

    def ddrain(j, carry):
        pltpu.make_async_copy(ones_v, dacc.at[dslab.at[0]], dsem).wait()
        return carry

    pltpu.sync_copy(zrows.at[pl.ds(s * TS, TS)], acc.at[pl.ds(s * TS, TS)])
    pltpu.sync_copy(zdeg.at[pl.ds(s * TS, TS)], dacc.at[pl.ds(s * TS, TS)])

    @pl.when(s < 15)
    def _():
        pltpu.sync_copy(fcat.at[pl.ds(c * N + s * TS, TS)],
                        ftab.at[pl.ds(s * TS, TS)])

    @pl.when(s == 15)
    def _():
        pltpu.sync_copy(fcat.at[pl.ds(c * N + 15 * TS, N - 15 * TS)],
                        ftab.at[pl.ds(15 * TS, N - 15 * TS)])

    plsc.subcore_barrier()

    for half in range(TB2 // HS):
        hbase = base + half * HS
        pltpu.sync_copy(srcb.at[pl.ds(hbase, HS)], sslab)
        pltpu.sync_copy(dstb.at[pl.ds(hbase, HS)], dslab)
        deg_here = c == (half // (TB2 // HS // 2))
        step = make_step(deg_here)

        for b in range(NB):
            gstart(b, b)

        def outer(g, carry):
            for b in range(NB):
                p = g * NB + b
                step(b, p)
                gstart(b, p + NB)
            return carry

        lax.fori_loop(0, PH // NB - 1, outer, 0)
        for b in range(NB):
            step(b, PH - NB + b)

        @pl.when(deg_here)
        def _():
            lax.fori_loop(0, PH, ddrain, 0)

    plsc.subcore_barrier()
    pltpu.sync_copy(acc.at[pl.ds(s * TS, TS)], outp.at[c, pl.ds(s * TS, TS)])
    pltpu.sync_copy(dacc.at[pl.ds(s * TS, TS)], outd.at[c, pl.ds(s * TS, TS)])


BR = 1000


def _enc_block(f_ref, p_ref, deg_ref, wa_ref, wbt_ref, wbb_ref, o_ref):
    f = f_ref[...]
    deg = jnp.maximum(deg_ref[:, 0:1] + deg_ref[:, 1:2], 1.0)
    acc = jnp.dot(f, wa_ref[...], preferred_element_type=jnp.float32)
    acc += jnp.dot(p_ref[0] / deg, wbt_ref[...], preferred_element_type=jnp.float32)
    acc += jnp.dot(p_ref[1] / deg, wbb_ref[...], preferred_element_type=jnp.float32)
    h = jnp.maximum(acc, 0.0)
    o_ref[0] = h[:, :HD]
    o_ref[1] = h[:, HD:]


def _encoder_tc(feats, partials, deg2, wa_t, wbt, wbb):
    return pl.pallas_call(
        _enc_block,
        grid=(N // BR,),
        in_specs=[
            pl.BlockSpec((BR, D), lambda i: (i, 0)),
            pl.BlockSpec((2, BR, HD), lambda i: (0, i, 0)),
            pl.BlockSpec((BR, 2), lambda i: (i, 0)),
            pl.BlockSpec((D, D), lambda i: (0, 0)),
            pl.BlockSpec((HD, D), lambda i: (0, 0)),
            pl.BlockSpec((HD, D), lambda i: (0, 0)),
        ],
        out_specs=pl.BlockSpec((2, BR, HD), lambda i: (0, i, 0)),
        out_shape=jax.ShapeDtypeStruct((2, N, HD), jnp.float32),
    )(feats, partials, deg2, wa_t, wbt, wbb)


def _final_block(h0_ref, p0_ref, g0_ref, h1_ref, p1_ref, g1_ref,
                 wat_ref, wab_ref, wbt_ref, wbb_ref, am_ref, o_ref):
    def enc(h_ref, p_ref, g_ref):
        deg = jnp.maximum(g_ref[:, 0:1] + g_ref[:, 1:2], 1.0)
        acc = jnp.dot(h_ref[0], wat_ref[...], preferred_element_type=jnp.float32)
        acc += jnp.dot(h_ref[1], wab_ref[...], preferred_element_type=jnp.float32)
        acc += jnp.dot(p_ref[0] / deg, wbt_ref[...],
                       preferred_element_type=jnp.float32)
        acc += jnp.dot(p_ref[1] / deg, wbb_ref[...],
                       preferred_element_type=jnp.float32)
        return jnp.maximum(acc, 0.0)

    e0 = enc(h0_ref, p0_ref, g0_ref)
    e1 = enc(h1_ref, p1_ref, g1_ref)
    am = am_ref[...]
    c0 = jnp.dot(e0, am, preferred_element_type=jnp.float32)
    c1 = jnp.dot(e1, am, preferred_element_type=jnp.float32)

    def lrelu(x):
        return jnp.where(x >= 0, x, 0.2 * x)

    s00 = lrelu(c0[:, 0:1] + c0[:, 1:2])
    s01 = lrelu(c0[:, 0:1] + c1[:, 1:2])
    s10 = lrelu(c1[:, 2:3] + c0[:, 3:4])
    s11 = lrelu(c1[:, 2:3] + c1[:, 3:4])
    m0 = jnp.maximum(s00, s01)
    w00 = jnp.exp(s00 - m0)
    w01 = jnp.exp(s01 - m0)
    o_ref[0] = (w00 * e0 + w01 * e1) / (w00 + w01)
    m1 = jnp.maximum(s10, s11)
    w10 = jnp.exp(s10 - m1)
    w11 = jnp.exp(s11 - m1)
    o_ref[1] = (w10 * e0 + w11 * e1) / (w10 + w11)


def _final_tc(h0, p0, g0, h1, p1, g1, wat, wab, wbt, wbb, am):
    split_spec = pl.BlockSpec((2, BR, HD), lambda i: (0, i, 0))
    deg_spec = pl.BlockSpec((BR, 2), lambda i: (i, 0))
    wa_spec = pl.BlockSpec((HD, D), lambda i: (0, 0))
    return pl.pallas_call(
        _final_block,
        grid=(N // BR,),
        in_specs=[
            split_spec, split_spec, deg_spec,
            split_spec, split_spec, deg_spec,
            wa_spec, wa_spec, wa_spec, wa_spec,
            pl.BlockSpec((D, 4), lambda i: (0, 0)),
        ],
        out_specs=pl.BlockSpec((2, BR, D), lambda i: (0, i, 0)),
        out_shape=jax.ShapeDtypeStruct((2, N, D), jnp.float32),
    )(h0, p0, g0, h1, p1, g1, wat, wab, wbt, wbb, am)


def _prep_edges(ei):
    src = ei[0].astype(jnp.int32)
    dst = ei[1].astype(jnp.int32)
    pad = EP - E
    src = jnp.concatenate([src, jnp.zeros((pad,), jnp.int32)]).reshape(RP2, SB)
    dst = jnp.concatenate([dst, jnp.full((pad,), N, jnp.int32)]).reshape(RP2, SB)
    return src, dst


def _split_cols(x):
    return jnp.concatenate([x[:, :HD], x[:, HD:]], axis=0)


def kernel(nodes, features0, features1, edge_index0, edge_index1, W1, W2, att):
    f0 = features0.astype(jnp.float32)
    f1 = features1.astype(jnp.float32)
    s0, d0 = _prep_edges(edge_index0)
    s1, d1 = _prep_edges(edge_index1)
    zrows = jnp.zeros((NACC, HD), jnp.float32)
    zdeg = jnp.zeros((NACC,), jnp.float32)
    w1a = W1[:, :D].T
    w1bt = W1[:, D:D + HD].T
    w1bb = W1[:, D + HD:].T
    w2at = W2[:, :HD].T
    w2ab = W2[:, HD:D].T
    w2bt = W2[:, D:D + HD].T
    w2bb = W2[:, D + HD:].T
    am = att.astype(jnp.float32).reshape(4, D).T

    def graph(feats, src, dst):
        p1, g = _seg_sum_sc(_split_cols(feats), src, dst, zrows, zdeg)
        g2 = g.T
        hs = _encoder_tc(feats, p1, g2, w1a, w1bt, w1bb)
        p2, _ = _seg_sum_sc(hs.reshape(2 * N, HD), src, dst, zrows, zdeg)
        return hs, p2, g2

    h0, p20, g0t = graph(f0, s0, d0)
    h1, p21, g1t = graph(f1, s1, d1)
    return _final_tc(h0, p20, g0t, h1, p21, g1t,
                     w2at, w2ab, w2bt, w2bb, am)
